# baseline (speedup 1.0000x reference)
import functools

import jax
import jax.numpy as jnp
from jax import lax
from jax.experimental import pallas as pl
from jax.experimental.pallas import tpu as pltpu
from jax.experimental.pallas import tpu_sc as plsc

N = 10000
E = 320000
D_IN = 128
D_HID = 16
NC = 2
NS = 16
NW = NC * NS
CHUNK = 128
CH = -(-E // (NW * CHUNK))
GCH = NW * CH
DCH = GCH // NS
EP = GCH * CHUNK
NP = NW * 320
RPW = NP // NS
TRASH = N
NBUF, LAG = 10, 5

_mesh = plsc.VectorSubcoreMesh(
    core_axis_name="c", subcore_axis_name="s", num_cores=NC, num_subcores=NS)

def _rsqrt16(d):
    i = plsc.bitcast(d, jnp.int32)
    y = plsc.bitcast(jnp.int32(0x5F3759DF) - (i >> 1), jnp.float32)
    for _ in range(3):
        y = y * (1.5 - 0.5 * d * y * y)
    return y


def _fill_rows(ref, rows):
    val16 = jnp.zeros((16,), jnp.float32)

    def body(i, _):
        ref[i, :] = val16
        return 0
    lax.fori_loop(0, rows, body, 0)


def _offset_indices(idx_ref, off):
    offv = jnp.zeros((16,), jnp.int32) + off

    def body(j, _):
        for k in range(CHUNK // 16):
            sl = pl.ds(k * 16, 16)
            idx_ref[j, sl] = idx_ref[j, sl] + offv
        return 0
    lax.fori_loop(0, CH, body, 0)


def _edge_pass(ht_hbm, src_v, dst_v, rows_v, acc_sh, gsems, ssems):
    gd = [None] * CH
    sd = [None] * CH
    for j in range(CH):
        b = j % NBUF
        if j >= NBUF:
            sd[j - NBUF].wait()
        gd[j] = pltpu.async_copy(ht_hbm.at[src_v.at[j]], rows_v.at[b],
                                 gsems[b])
        if j >= LAG:
            k = j - LAG
            gd[k].wait()
            sd[k] = pltpu.async_copy(rows_v.at[k % NBUF],
                                     acc_sh.at[dst_v.at[k]],
                                     ssems[k % NBUF], add=True)
    for k in range(CH - LAG, CH):
        gd[k].wait()
        sd[k] = pltpu.async_copy(rows_v.at[k % NBUF],
                                 acc_sh.at[dst_v.at[k]],
                                 ssems[k % NBUF], add=True)
    for k in range(CH - NBUF, CH):
        sd[k].wait()


@functools.partial(
    pl.kernel,
    out_type=(jax.ShapeDtypeStruct((NC, NP, D_HID), jnp.float32),
              jax.ShapeDtypeStruct((NC * NP, D_HID), jnp.float32),
              jax.ShapeDtypeStruct((NC * NP // 16, 16), jnp.float32)),
    mesh=_mesh,
    scratch_types=[
        pltpu.VMEM((DCH, CHUNK), jnp.int32),
        pltpu.VMEM((CH, CHUNK), jnp.int32),
        pltpu.VMEM((CH, CHUNK), jnp.int32),
        pltpu.VMEM((RPW, D_HID), jnp.float32),
        pltpu.VMEM((RPW,), jnp.float32),
        pltpu.VMEM((RPW // 16, 16), jnp.float32),
        pltpu.VMEM((RPW,), jnp.float32),
        pltpu.VMEM((CHUNK,), jnp.float32),
        pltpu.VMEM((CHUNK, D_HID), jnp.float32),
        pltpu.VMEM((NBUF, CHUNK, D_HID), jnp.float32),
        pltpu.VMEM_SHARED((NP,), jnp.float32),
        pltpu.VMEM_SHARED((NP, D_HID), jnp.float32),
        [pltpu.SemaphoreType.DMA] * 8,
        [pltpu.SemaphoreType.DMA] * NBUF,
        [pltpu.SemaphoreType.DMA] * NBUF,
        pltpu.SemaphoreType.DMA,
    ],
    compiler_params=pltpu.CompilerParams(use_tc_tiling_on_sc=False, needs_layout_passes=False),
)
def _sc1_kernel(h1_hbm, src_hbm, dst_hbm, acc_out, ht_out, dis_out,
                ddst_v, src_v, dst_v, lrows_v, ldeg_v, ldis_v, zrow_v,
                ones_v, ztile_v, rows_v, deg_sh, acc_sh, dsems, gsems,
                ssems, hsem):
    cid = lax.axis_index("c")
    sid = lax.axis_index("s")
    w = cid * NS + sid
    nbase = sid * RPW

    hdesc = pltpu.async_copy(h1_hbm.at[pl.ds(nbase, RPW)], lrows_v, hsem)
    pltpu.sync_copy(dst_hbm.at[pl.ds(sid * DCH, DCH)], ddst_v)
    pltpu.sync_copy(src_hbm.at[pl.ds(w * CH, CH)], src_v)
    pltpu.sync_copy(dst_hbm.at[pl.ds(w * CH, CH)], dst_v)

    one16 = jnp.ones((16,), jnp.float32)
    for i in range(CHUNK // 16):
        ones_v[pl.ds(i * 16, 16)] = one16
    for g in range(RPW // 16):
        zrow_v[pl.ds(g * 16, 16)] = jnp.zeros((16,), jnp.float32)
    _fill_rows(ztile_v, CHUNK)
    pltpu.sync_copy(zrow_v, deg_sh.at[pl.ds(nbase, RPW)])
    for t in range(RPW // CHUNK):
        pltpu.sync_copy(ztile_v, acc_sh.at[pl.ds(nbase + t * CHUNK, CHUNK)])
    _offset_indices(src_v, cid * NP)
    plsc.subcore_barrier()

    dd = [None] * DCH
    for j in range(DCH):
        if j >= 8:
            dd[j - 8].wait()
        dd[j] = pltpu.async_copy(ones_v, deg_sh.at[ddst_v.at[j]],
                                 dsems[j % 8], add=True)
    for j in range(DCH - 8, DCH):
        dd[j].wait()
    plsc.subcore_barrier()

    pltpu.sync_copy(deg_sh.at[pl.ds(nbase, RPW)], ldeg_v)
    hdesc.wait()
    for g in range(RPW // 16):
        ldis_v[g, :] = _rsqrt16(ldeg_v[pl.ds(g * 16, 16)] + 1.0)

    def scale_body(g, _):
        dvec = ldis_v[g, :]
        for k in range(16):
            r = g * 16 + k
            lrows_v[r, :] = lrows_v[r, :] * dvec[k]
        return 0
    lax.fori_loop(0, RPW // 16, scale_body, 0)
    pltpu.sync_copy(lrows_v, ht_out.at[pl.ds(cid * NP + nbase, RPW)])
    pltpu.sync_copy(
        ldis_v, dis_out.at[pl.ds(cid * (NP // 16) + sid * (RPW // 16),
                                 RPW // 16)])
    plsc.subcore_barrier()

    _edge_pass(ht_out, src_v, dst_v, rows_v, acc_sh, gsems, ssems)
    plsc.subcore_barrier()
    pltpu.sync_copy(acc_sh.at[pl.ds(nbase, RPW)],
                    acc_out.at[cid, pl.ds(nbase, RPW)])


@functools.partial(
    pl.kernel,
    out_type=(jax.ShapeDtypeStruct((NC, NP, D_HID), jnp.float32),
              jax.ShapeDtypeStruct((NC * NP, D_HID), jnp.float32)),
    mesh=_mesh,
    scratch_types=[
        pltpu.VMEM((CH, CHUNK), jnp.int32),
        pltpu.VMEM((CH, CHUNK), jnp.int32),
        pltpu.VMEM((RPW, D_HID), jnp.float32),
        pltpu.VMEM((RPW, D_HID), jnp.float32),
        pltpu.VMEM((RPW, D_HID), jnp.float32),
        pltpu.VMEM((RPW // 16, 16), jnp.float32),
        pltpu.VMEM((16,), jnp.float32),
        pltpu.VMEM((CHUNK, D_HID), jnp.float32),
        pltpu.VMEM((NBUF, CHUNK, D_HID), jnp.float32),
        pltpu.VMEM_SHARED((NP, D_HID), jnp.float32),
        [pltpu.SemaphoreType.DMA] * NBUF,
        [pltpu.SemaphoreType.DMA] * NBUF,
    ],
    compiler_params=pltpu.CompilerParams(use_tc_tiling_on_sc=False, needs_layout_passes=False),
)
def _sc2_kernel(acc1_hbm, ht1_hbm, dis_hbm, b1_hbm, src_hbm, dst_hbm,
                acc_out, ht_out,
                src_v, dst_v, lrows_v, lacc_v, lacc2_v, ldis_v, b1_v,
                ztile_v, rows_v, acc_sh, gsems, ssems):
    cid = lax.axis_index("c")
    sid = lax.axis_index("s")
    w = cid * NS + sid
    nbase = sid * RPW

    pltpu.sync_copy(src_hbm.at[pl.ds(w * CH, CH)], src_v)
    pltpu.sync_copy(dst_hbm.at[pl.ds(w * CH, CH)], dst_v)
    pltpu.sync_copy(acc1_hbm.at[0, pl.ds(nbase, RPW)], lacc_v)
    pltpu.sync_copy(acc1_hbm.at[1, pl.ds(nbase, RPW)], lacc2_v)
    pltpu.sync_copy(ht1_hbm.at[pl.ds(cid * NP + nbase, RPW)], lrows_v)
    pltpu.sync_copy(
        dis_hbm.at[pl.ds(cid * (NP // 16) + sid * (RPW // 16), RPW // 16)],
        ldis_v)
    pltpu.sync_copy(b1_hbm, b1_v)
    _fill_rows(ztile_v, CHUNK)
    for t in range(RPW // CHUNK):
        pltpu.sync_copy(ztile_v, acc_sh.at[pl.ds(nbase + t * CHUNK, CHUNK)])
    _offset_indices(src_v, cid * NP)

    b1vec = b1_v[...]

    def relu_body(g, _):
        dvec = ldis_v[g, :]
        for k in range(16):
            r = g * 16 + k
            s = dvec[k]
            hr = jnp.maximum(
                s * (lacc_v[r, :] + lacc2_v[r, :] + lrows_v[r, :]) + b1vec,
                0.0)
            lrows_v[r, :] = s * hr
        return 0
    lax.fori_loop(0, RPW // 16, relu_body, 0)
    pltpu.sync_copy(lrows_v, ht_out.at[pl.ds(cid * NP + nbase, RPW)])
    plsc.subcore_barrier()

    _edge_pass(ht_out, src_v, dst_v, rows_v, acc_sh, gsems, ssems)
    plsc.subcore_barrier()
    pltpu.sync_copy(acc_sh.at[pl.ds(nbase, RPW)],
                    acc_out.at[cid, pl.ds(nbase, RPW)])


def _mm1_body(x_ref, w1_ref, h_ref):
    h_ref[...] = jnp.dot(x_ref[...], w1_ref[...],
                         preferred_element_type=jnp.float32)


def _final_body(accp_ref, ht2_ref, dis_ref, w2_ref, b2_ref, out_ref):
    acc = accp_ref[0] + accp_ref[1]
    agg = dis_ref[...][:, None] * (acc + ht2_ref[...][:NP])
    out_ref[...] = (
        jnp.dot(agg[:N], w2_ref[...], preferred_element_type=jnp.float32)
        + b2_ref[...][None, :])


def kernel(x, edge_index, W1, b1, W2, b2):
    src = edge_index[0]
    dst = edge_index[1]
    pad = EP - E
    src_p = jnp.concatenate([src, jnp.zeros((pad,), jnp.int32)]).reshape(
        GCH, CHUNK)
    dst_p = jnp.concatenate([dst, jnp.full((pad,), TRASH, jnp.int32)]).reshape(
        GCH, CHUNK)
    x_p = jnp.concatenate([x, jnp.zeros((NP - N, D_IN), jnp.float32)])

    h1 = pl.pallas_call(
        _mm1_body,
        out_shape=jax.ShapeDtypeStruct((NP, D_HID), jnp.float32),
    )(x_p, W1)

    acc1, ht1, dis = _sc1_kernel(h1, src_p, dst_p)
    acc2, ht2 = _sc2_kernel(acc1, ht1, dis, b1, src_p, dst_p)

    dis_flat = dis.reshape(NC * NP)[:NP]
    out = pl.pallas_call(
        _final_body,
        out_shape=jax.ShapeDtypeStruct((N, D_IN), jnp.float32),
    )(acc2, ht2, dis_flat, W2, b2)
    return out


# speedup vs baseline: 1.2753x; 1.2753x over previous
import functools

import jax
import jax.numpy as jnp
from jax import lax
from jax.experimental import pallas as pl
from jax.experimental.pallas import tpu as pltpu
from jax.experimental.pallas import tpu_sc as plsc

N = 10000
E = 320000
D_IN = 128
D_HID = 16
NC = 2
NS = 16
NW = NC * NS
CHUNK = 128
CH = -(-E // (NW * CHUNK))
GCH = NW * CH
DCH = GCH // NS
EP = GCH * CHUNK
NP = NW * 320
RPW = NP // NS
TRASH = N
NBUF, LAG = 10, 5

_mesh = plsc.VectorSubcoreMesh(
    core_axis_name="c", subcore_axis_name="s", num_cores=NC, num_subcores=NS)

def _rsqrt16(d):
    i = plsc.bitcast(d, jnp.int32)
    y = plsc.bitcast(jnp.int32(0x5F3759DF) - (i >> 1), jnp.float32)
    for _ in range(3):
        y = y * (1.5 - 0.5 * d * y * y)
    return y


def _fill_rows(ref, rows):
    val16 = jnp.zeros((16,), jnp.float32)

    def body(i, _):
        ref[i, :] = val16
        return 0
    lax.fori_loop(0, rows, body, 0)


def _offset_indices(idx_ref, off):
    offv = jnp.zeros((16,), jnp.int32) + off

    def body(j, _):
        for k in range(CHUNK // 16):
            sl = pl.ds(k * 16, 16)
            idx_ref[j, sl] = idx_ref[j, sl] + offv
        return 0
    lax.fori_loop(0, CH, body, 0)


def _edge_pass(ht_hbm, src_v, dst_v, rows_v, acc_sh, gsems, ssems):
    gd = [None] * CH
    sd = [None] * CH
    for j in range(CH):
        b = j % NBUF
        if j >= NBUF:
            sd[j - NBUF].wait()
        gd[j] = pltpu.async_copy(ht_hbm.at[src_v.at[j]], rows_v.at[b],
                                 gsems[b])
        if j >= LAG:
            k = j - LAG
            gd[k].wait()
            sd[k] = pltpu.async_copy(rows_v.at[k % NBUF],
                                     acc_sh.at[dst_v.at[k]],
                                     ssems[k % NBUF], add=True)
    for k in range(CH - LAG, CH):
        gd[k].wait()
        sd[k] = pltpu.async_copy(rows_v.at[k % NBUF],
                                 acc_sh.at[dst_v.at[k]],
                                 ssems[k % NBUF], add=True)
    for k in range(CH - NBUF, CH):
        sd[k].wait()


@functools.partial(
    pl.kernel,
    out_type=(jax.ShapeDtypeStruct((NC, NP, D_HID), jnp.float32),
              jax.ShapeDtypeStruct((NC * NP, D_HID), jnp.float32),
              jax.ShapeDtypeStruct((NC * NP // 16, 16), jnp.float32)),
    mesh=_mesh,
    scratch_types=[
        pltpu.VMEM((DCH, CHUNK), jnp.int32),
        pltpu.VMEM((CH, CHUNK), jnp.int32),
        pltpu.VMEM((CH, CHUNK), jnp.int32),
        pltpu.VMEM((RPW, D_HID), jnp.float32),
        pltpu.VMEM((RPW,), jnp.float32),
        pltpu.VMEM((RPW // 16, 16), jnp.float32),
        pltpu.VMEM((RPW,), jnp.float32),
        pltpu.VMEM((CHUNK,), jnp.float32),
        pltpu.VMEM((CHUNK, D_HID), jnp.float32),
        pltpu.VMEM((NBUF, CHUNK, D_HID), jnp.float32),
        pltpu.VMEM_SHARED((NP,), jnp.float32),
        pltpu.VMEM_SHARED((NP, D_HID), jnp.float32),
        pltpu.VMEM_SHARED((NP, D_HID), jnp.float32),
        [pltpu.SemaphoreType.DMA] * 8,
        [pltpu.SemaphoreType.DMA] * NBUF,
        [pltpu.SemaphoreType.DMA] * NBUF,
        pltpu.SemaphoreType.DMA,
        pltpu.SemaphoreType.DMA,
    ],
    compiler_params=pltpu.CompilerParams(use_tc_tiling_on_sc=False, needs_layout_passes=False),
)
def _sc1_kernel(h1_hbm, src_hbm, dst_hbm, acc_out, ht_out, dis_out,
                ddst_v, src_v, dst_v, lrows_v, ldeg_v, ldis_v, zrow_v,
                ones_v, ztile_v, rows_v, deg_sh, acc_sh, ht_sh, dsems,
                gsems, ssems, hsem, hsem2):
    cid = lax.axis_index("c")
    sid = lax.axis_index("s")
    w = cid * NS + sid
    nbase = sid * RPW

    hdesc = pltpu.async_copy(h1_hbm.at[pl.ds(nbase, RPW)], lrows_v, hsem)
    pltpu.sync_copy(dst_hbm.at[pl.ds(sid * DCH, DCH)], ddst_v)
    pltpu.sync_copy(src_hbm.at[pl.ds(w * CH, CH)], src_v)
    pltpu.sync_copy(dst_hbm.at[pl.ds(w * CH, CH)], dst_v)

    one16 = jnp.ones((16,), jnp.float32)
    for i in range(CHUNK // 16):
        ones_v[pl.ds(i * 16, 16)] = one16
    for g in range(RPW // 16):
        zrow_v[pl.ds(g * 16, 16)] = jnp.zeros((16,), jnp.float32)
    _fill_rows(ztile_v, CHUNK)
    pltpu.sync_copy(zrow_v, deg_sh.at[pl.ds(nbase, RPW)])
    for t in range(RPW // CHUNK):
        pltpu.sync_copy(ztile_v, acc_sh.at[pl.ds(nbase + t * CHUNK, CHUNK)])
    plsc.subcore_barrier()

    dd = [None] * DCH
    for j in range(DCH):
        if j >= 8:
            dd[j - 8].wait()
        dd[j] = pltpu.async_copy(ones_v, deg_sh.at[ddst_v.at[j]],
                                 dsems[j % 8], add=True)
    for j in range(DCH - 8, DCH):
        dd[j].wait()
    plsc.subcore_barrier()

    pltpu.sync_copy(deg_sh.at[pl.ds(nbase, RPW)], ldeg_v)
    hdesc.wait()
    for g in range(RPW // 16):
        ldis_v[g, :] = _rsqrt16(ldeg_v[pl.ds(g * 16, 16)] + 1.0)

    def scale_body(g, _):
        dvec = ldis_v[g, :]
        for k in range(16):
            r = g * 16 + k
            lrows_v[r, :] = lrows_v[r, :] * dvec[k]
        return 0
    lax.fori_loop(0, RPW // 16, scale_body, 0)
    pltpu.sync_copy(lrows_v, ht_sh.at[pl.ds(nbase, RPW)])
    hd2 = pltpu.async_copy(lrows_v, ht_out.at[pl.ds(cid * NP + nbase, RPW)],
                           hsem2)
    pltpu.sync_copy(
        ldis_v, dis_out.at[pl.ds(cid * (NP // 16) + sid * (RPW // 16),
                                 RPW // 16)])
    plsc.subcore_barrier()

    _edge_pass(ht_sh, src_v, dst_v, rows_v, acc_sh, gsems, ssems)
    hd2.wait()
    plsc.subcore_barrier()
    pltpu.sync_copy(acc_sh.at[pl.ds(nbase, RPW)],
                    acc_out.at[cid, pl.ds(nbase, RPW)])


@functools.partial(
    pl.kernel,
    out_type=(jax.ShapeDtypeStruct((NC, NP, D_HID), jnp.float32),
              jax.ShapeDtypeStruct((NC * NP, D_HID), jnp.float32)),
    mesh=_mesh,
    scratch_types=[
        pltpu.VMEM((CH, CHUNK), jnp.int32),
        pltpu.VMEM((CH, CHUNK), jnp.int32),
        pltpu.VMEM((RPW, D_HID), jnp.float32),
        pltpu.VMEM((RPW, D_HID), jnp.float32),
        pltpu.VMEM((RPW, D_HID), jnp.float32),
        pltpu.VMEM((RPW // 16, 16), jnp.float32),
        pltpu.VMEM((16,), jnp.float32),
        pltpu.VMEM((CHUNK, D_HID), jnp.float32),
        pltpu.VMEM((NBUF, CHUNK, D_HID), jnp.float32),
        pltpu.VMEM_SHARED((NP, D_HID), jnp.float32),
        pltpu.VMEM_SHARED((NP, D_HID), jnp.float32),
        [pltpu.SemaphoreType.DMA] * NBUF,
        [pltpu.SemaphoreType.DMA] * NBUF,
        pltpu.SemaphoreType.DMA,
    ],
    compiler_params=pltpu.CompilerParams(use_tc_tiling_on_sc=False, needs_layout_passes=False),
)
def _sc2_kernel(acc1_hbm, ht1_hbm, dis_hbm, b1_hbm, src_hbm, dst_hbm,
                acc_out, ht_out,
                src_v, dst_v, lrows_v, lacc_v, lacc2_v, ldis_v, b1_v,
                ztile_v, rows_v, acc_sh, ht_sh, gsems, ssems, hsem2):
    cid = lax.axis_index("c")
    sid = lax.axis_index("s")
    w = cid * NS + sid
    nbase = sid * RPW

    pltpu.sync_copy(src_hbm.at[pl.ds(w * CH, CH)], src_v)
    pltpu.sync_copy(dst_hbm.at[pl.ds(w * CH, CH)], dst_v)
    pltpu.sync_copy(acc1_hbm.at[0, pl.ds(nbase, RPW)], lacc_v)
    pltpu.sync_copy(acc1_hbm.at[1, pl.ds(nbase, RPW)], lacc2_v)
    pltpu.sync_copy(ht1_hbm.at[pl.ds(cid * NP + nbase, RPW)], lrows_v)
    pltpu.sync_copy(
        dis_hbm.at[pl.ds(cid * (NP // 16) + sid * (RPW // 16), RPW // 16)],
        ldis_v)
    pltpu.sync_copy(b1_hbm, b1_v)
    _fill_rows(ztile_v, CHUNK)
    for t in range(RPW // CHUNK):
        pltpu.sync_copy(ztile_v, acc_sh.at[pl.ds(nbase + t * CHUNK, CHUNK)])

    b1vec = b1_v[...]

    def relu_body(g, _):
        dvec = ldis_v[g, :]
        for k in range(16):
            r = g * 16 + k
            s = dvec[k]
            hr = jnp.maximum(
                s * (lacc_v[r, :] + lacc2_v[r, :] + lrows_v[r, :]) + b1vec,
                0.0)
            lrows_v[r, :] = s * hr
        return 0
    lax.fori_loop(0, RPW // 16, relu_body, 0)
    pltpu.sync_copy(lrows_v, ht_sh.at[pl.ds(nbase, RPW)])
    hd2 = pltpu.async_copy(lrows_v, ht_out.at[pl.ds(cid * NP + nbase, RPW)],
                           hsem2)
    plsc.subcore_barrier()

    _edge_pass(ht_sh, src_v, dst_v, rows_v, acc_sh, gsems, ssems)
    hd2.wait()
    plsc.subcore_barrier()
    pltpu.sync_copy(acc_sh.at[pl.ds(nbase, RPW)],
                    acc_out.at[cid, pl.ds(nbase, RPW)])


def _mm1_body(x_ref, w1_ref, h_ref):
    h_ref[...] = jnp.dot(x_ref[...], w1_ref[...],
                         preferred_element_type=jnp.float32)


def _final_body(accp_ref, ht2_ref, dis_ref, w2_ref, b2_ref, out_ref):
    acc = accp_ref[0] + accp_ref[1]
    agg = dis_ref[...][:, None] * (acc + ht2_ref[...][:NP])
    out_ref[...] = (
        jnp.dot(agg[:N], w2_ref[...], preferred_element_type=jnp.float32)
        + b2_ref[...][None, :])


def kernel(x, edge_index, W1, b1, W2, b2):
    src = edge_index[0]
    dst = edge_index[1]
    pad = EP - E
    src_p = jnp.concatenate([src, jnp.zeros((pad,), jnp.int32)]).reshape(
        GCH, CHUNK)
    dst_p = jnp.concatenate([dst, jnp.full((pad,), TRASH, jnp.int32)]).reshape(
        GCH, CHUNK)
    x_p = jnp.concatenate([x, jnp.zeros((NP - N, D_IN), jnp.float32)])

    h1 = pl.pallas_call(
        _mm1_body,
        out_shape=jax.ShapeDtypeStruct((NP, D_HID), jnp.float32),
    )(x_p, W1)

    acc1, ht1, dis = _sc1_kernel(h1, src_p, dst_p)
    acc2, ht2 = _sc2_kernel(acc1, ht1, dis, b1, src_p, dst_p)

    dis_flat = dis.reshape(NC * NP)[:NP]
    out = pl.pallas_call(
        _final_body,
        out_shape=jax.ShapeDtypeStruct((N, D_IN), jnp.float32),
    )(acc2, ht2, dis_flat, W2, b2)
    return out


# speedup vs baseline: 1.3576x; 1.0645x over previous
import functools

import jax
import jax.numpy as jnp
from jax import lax
from jax.experimental import pallas as pl
from jax.experimental.pallas import tpu as pltpu
from jax.experimental.pallas import tpu_sc as plsc

N = 10000
E = 320000
D_IN = 128
D_HID = 16
NC = 2
NS = 16
CHUNK = 128
GCH = E // CHUNK
PCH = GCH // NC
WCH = PCH // NS
PREM = PCH - NS * WCH
DCH = GCH // NS
DREM = GCH - NS * DCH
NP = 10240
RPW = NP // NS
NBUF, LAG = 10, 5

_mesh = plsc.VectorSubcoreMesh(
    core_axis_name="c", subcore_axis_name="s", num_cores=NC, num_subcores=NS)

_SC_PARAMS = pltpu.CompilerParams(
    use_tc_tiling_on_sc=False, needs_layout_passes=False)


def _rsqrt16(d):
    i = plsc.bitcast(d, jnp.int32)
    y = plsc.bitcast(jnp.int32(0x5F3759DF) - (i >> 1), jnp.float32)
    for _ in range(3):
        y = y * (1.5 - 0.5 * d * y * y)
    return y


def _fill_rows(ref, rows):
    val16 = jnp.zeros((16,), jnp.float32)

    def body(i, _):
        ref[i, :] = val16
        return 0
    lax.fori_loop(0, rows, body, 0)


def _edge_pass(ht_sh, src_v, dst_v, rows_v, acc_sh, gsems, ssems, extra):
    gd = [None] * WCH
    sd = [None] * WCH
    for j in range(WCH):
        b = j % NBUF
        if j >= NBUF:
            sd[j - NBUF].wait()
        gd[j] = pltpu.async_copy(ht_sh.at[src_v.at[j]], rows_v.at[b],
                                 gsems[b])
        if j >= LAG:
            k = j - LAG
            gd[k].wait()
            sd[k] = pltpu.async_copy(rows_v.at[k % NBUF],
                                     acc_sh.at[dst_v.at[k]],
                                     ssems[k % NBUF], add=True)
    for k in range(WCH - LAG, WCH):
        gd[k].wait()
        sd[k] = pltpu.async_copy(rows_v.at[k % NBUF],
                                 acc_sh.at[dst_v.at[k]],
                                 ssems[k % NBUF], add=True)
    for k in range(WCH - NBUF, WCH):
        sd[k].wait()

    @pl.when(extra)
    def _():
        pltpu.sync_copy(ht_sh.at[src_v.at[WCH]], rows_v.at[0])
        pltpu.sync_copy(rows_v.at[0], acc_sh.at[dst_v.at[WCH]], add=True)


def _load_worker_chunks(src_hbm, dst_hbm, src_v, dst_v, cid, sid):
    base = cid * PCH + sid * WCH
    xrow = cid * PCH + NS * WCH + lax.rem(sid, PREM)
    pltpu.sync_copy(src_hbm.at[pl.ds(base, WCH)], src_v.at[pl.ds(0, WCH)])
    pltpu.sync_copy(dst_hbm.at[pl.ds(base, WCH)], dst_v.at[pl.ds(0, WCH)])
    pltpu.sync_copy(src_hbm.at[pl.ds(xrow, 1)], src_v.at[pl.ds(WCH, 1)])
    pltpu.sync_copy(dst_hbm.at[pl.ds(xrow, 1)], dst_v.at[pl.ds(WCH, 1)])


@functools.partial(
    pl.kernel,
    out_type=(jax.ShapeDtypeStruct((NC, NP, D_HID), jnp.float32),
              jax.ShapeDtypeStruct((NC * NP, D_HID), jnp.float32),
              jax.ShapeDtypeStruct((NC * NP // 16, 16), jnp.float32)),
    mesh=_mesh,
    scratch_types=[
        pltpu.VMEM((DCH + 1, CHUNK), jnp.int32),
        pltpu.VMEM((WCH + 1, CHUNK), jnp.int32),
        pltpu.VMEM((WCH + 1, CHUNK), jnp.int32),
        pltpu.VMEM((RPW, D_HID), jnp.float32),
        pltpu.VMEM((RPW,), jnp.float32),
        pltpu.VMEM((RPW // 16, 16), jnp.float32),
        pltpu.VMEM((RPW,), jnp.float32),
        pltpu.VMEM((CHUNK,), jnp.float32),
        pltpu.VMEM((CHUNK, D_HID), jnp.float32),
        pltpu.VMEM((NBUF, CHUNK, D_HID), jnp.float32),
        pltpu.VMEM_SHARED((NP,), jnp.float32),
        pltpu.VMEM_SHARED((NP, D_HID), jnp.float32),
        pltpu.VMEM_SHARED((NP, D_HID), jnp.float32),
        [pltpu.SemaphoreType.DMA] * 8,
        [pltpu.SemaphoreType.DMA] * NBUF,
        [pltpu.SemaphoreType.DMA] * NBUF,
        pltpu.SemaphoreType.DMA,
        pltpu.SemaphoreType.DMA,
    ],
    compiler_params=_SC_PARAMS,
)
def _sc1_kernel(h1_hbm, src_hbm, dst_hbm, acc_out, ht_out, dis_out,
                ddst_v, src_v, dst_v, lrows_v, ldeg_v, ldis_v, zrow_v,
                ones_v, ztile_v, rows_v, deg_sh, acc_sh, ht_sh, dsems,
                gsems, ssems, hsem, hsem2):
    cid = lax.axis_index("c")
    sid = lax.axis_index("s")
    nbase = sid * RPW

    hdesc = pltpu.async_copy(h1_hbm.at[pl.ds(nbase, RPW)], lrows_v, hsem)
    pltpu.sync_copy(dst_hbm.at[pl.ds(sid * DCH, DCH)],
                    ddst_v.at[pl.ds(0, DCH)])
    pltpu.sync_copy(dst_hbm.at[pl.ds(NS * DCH + lax.rem(sid, DREM), 1)],
                    ddst_v.at[pl.ds(DCH, 1)])
    _load_worker_chunks(src_hbm, dst_hbm, src_v, dst_v, cid, sid)

    one16 = jnp.ones((16,), jnp.float32)
    for i in range(CHUNK // 16):
        ones_v[pl.ds(i * 16, 16)] = one16
    for g in range(RPW // 16):
        zrow_v[pl.ds(g * 16, 16)] = jnp.zeros((16,), jnp.float32)
    _fill_rows(ztile_v, CHUNK)
    pltpu.sync_copy(zrow_v, deg_sh.at[pl.ds(nbase, RPW)])
    for t in range(RPW // CHUNK):
        pltpu.sync_copy(ztile_v, acc_sh.at[pl.ds(nbase + t * CHUNK, CHUNK)])
    plsc.subcore_barrier()

    dd = [None] * DCH
    for j in range(DCH):
        if j >= 8:
            dd[j - 8].wait()
        dd[j] = pltpu.async_copy(ones_v, deg_sh.at[ddst_v.at[j]],
                                 dsems[j % 8], add=True)
    for j in range(DCH - 8, DCH):
        dd[j].wait()

    @pl.when(sid < DREM)
    def _():
        pltpu.sync_copy(ones_v, deg_sh.at[ddst_v.at[DCH]], add=True)
    plsc.subcore_barrier()

    pltpu.sync_copy(deg_sh.at[pl.ds(nbase, RPW)], ldeg_v)
    hdesc.wait()
    for g in range(RPW // 16):
        ldis_v[g, :] = _rsqrt16(ldeg_v[pl.ds(g * 16, 16)] + 1.0)

    def scale_body(g, _):
        dvec = ldis_v[g, :]
        for k in range(16):
            r = g * 16 + k
            lrows_v[r, :] = lrows_v[r, :] * dvec[k]
        return 0
    lax.fori_loop(0, RPW // 16, scale_body, 0)
    pltpu.sync_copy(lrows_v, ht_sh.at[pl.ds(nbase, RPW)])
    hd2 = pltpu.async_copy(lrows_v, ht_out.at[pl.ds(cid * NP + nbase, RPW)],
                           hsem2)
    pltpu.sync_copy(
        ldis_v, dis_out.at[pl.ds(cid * (NP // 16) + sid * (RPW // 16),
                                 RPW // 16)])
    plsc.subcore_barrier()

    _edge_pass(ht_sh, src_v, dst_v, rows_v, acc_sh, gsems, ssems,
               sid < PREM)
    hd2.wait()
    plsc.subcore_barrier()
    pltpu.sync_copy(acc_sh.at[pl.ds(nbase, RPW)],
                    acc_out.at[cid, pl.ds(nbase, RPW)])


@functools.partial(
    pl.kernel,
    out_type=(jax.ShapeDtypeStruct((NC, NP, D_HID), jnp.float32),
              jax.ShapeDtypeStruct((NC * NP, D_HID), jnp.float32)),
    mesh=_mesh,
    scratch_types=[
        pltpu.VMEM((WCH + 1, CHUNK), jnp.int32),
        pltpu.VMEM((WCH + 1, CHUNK), jnp.int32),
        pltpu.VMEM((RPW, D_HID), jnp.float32),
        pltpu.VMEM((RPW, D_HID), jnp.float32),
        pltpu.VMEM((RPW, D_HID), jnp.float32),
        pltpu.VMEM((RPW // 16, 16), jnp.float32),
        pltpu.VMEM((16,), jnp.float32),
        pltpu.VMEM((CHUNK, D_HID), jnp.float32),
        pltpu.VMEM((NBUF, CHUNK, D_HID), jnp.float32),
        pltpu.VMEM_SHARED((NP, D_HID), jnp.float32),
        pltpu.VMEM_SHARED((NP, D_HID), jnp.float32),
        [pltpu.SemaphoreType.DMA] * NBUF,
        [pltpu.SemaphoreType.DMA] * NBUF,
        pltpu.SemaphoreType.DMA,
    ],
    compiler_params=_SC_PARAMS,
)
def _sc2_kernel(acc1_hbm, ht1_hbm, dis_hbm, b1_hbm, src_hbm, dst_hbm,
                acc_out, ht_out,
                src_v, dst_v, lrows_v, lacc_v, lacc2_v, ldis_v, b1_v,
                ztile_v, rows_v, acc_sh, ht_sh, gsems, ssems, hsem2):
    cid = lax.axis_index("c")
    sid = lax.axis_index("s")
    nbase = sid * RPW

    _load_worker_chunks(src_hbm, dst_hbm, src_v, dst_v, cid, sid)
    pltpu.sync_copy(acc1_hbm.at[0, pl.ds(nbase, RPW)], lacc_v)
    pltpu.sync_copy(acc1_hbm.at[1, pl.ds(nbase, RPW)], lacc2_v)
    pltpu.sync_copy(ht1_hbm.at[pl.ds(cid * NP + nbase, RPW)], lrows_v)
    pltpu.sync_copy(
        dis_hbm.at[pl.ds(cid * (NP // 16) + sid * (RPW // 16), RPW // 16)],
        ldis_v)
    pltpu.sync_copy(b1_hbm, b1_v)
    _fill_rows(ztile_v, CHUNK)
    for t in range(RPW // CHUNK):
        pltpu.sync_copy(ztile_v, acc_sh.at[pl.ds(nbase + t * CHUNK, CHUNK)])

    b1vec = b1_v[...]

    def relu_body(g, _):
        dvec = ldis_v[g, :]
        for k in range(16):
            r = g * 16 + k
            s = dvec[k]
            hr = jnp.maximum(
                s * (lacc_v[r, :] + lacc2_v[r, :] + lrows_v[r, :]) + b1vec,
                0.0)
            lrows_v[r, :] = s * hr
        return 0
    lax.fori_loop(0, RPW // 16, relu_body, 0)
    pltpu.sync_copy(lrows_v, ht_sh.at[pl.ds(nbase, RPW)])
    hd2 = pltpu.async_copy(lrows_v, ht_out.at[pl.ds(cid * NP + nbase, RPW)],
                           hsem2)
    plsc.subcore_barrier()

    _edge_pass(ht_sh, src_v, dst_v, rows_v, acc_sh, gsems, ssems,
               sid < PREM)
    hd2.wait()
    plsc.subcore_barrier()
    pltpu.sync_copy(acc_sh.at[pl.ds(nbase, RPW)],
                    acc_out.at[cid, pl.ds(nbase, RPW)])


def _mm1_body(x_ref, w1_ref, h_ref):
    h_ref[:N, :] = jnp.dot(x_ref[...], w1_ref[...],
                           preferred_element_type=jnp.float32)
    h_ref[N:, :] = jnp.zeros((NP - N, D_HID), jnp.float32)


def _final_body(accp_ref, ht2_ref, dis_ref, w2_ref, b2_ref, out_ref):
    acc = accp_ref[0] + accp_ref[1]
    agg = dis_ref[...][:, None] * (acc + ht2_ref[...][:NP])
    out_ref[...] = (
        jnp.dot(agg[:N], w2_ref[...], preferred_element_type=jnp.float32)
        + b2_ref[...][None, :])


def kernel(x, edge_index, W1, b1, W2, b2):
    src2 = edge_index[0].reshape(GCH, CHUNK)
    dst2 = edge_index[1].reshape(GCH, CHUNK)

    h1 = pl.pallas_call(
        _mm1_body,
        out_shape=jax.ShapeDtypeStruct((NP, D_HID), jnp.float32),
    )(x, W1)

    acc1, ht1, dis = _sc1_kernel(h1, src2, dst2)
    acc2, ht2 = _sc2_kernel(acc1, ht1, dis, b1, src2, dst2)

    dis_flat = dis.reshape(NC * NP)[:NP]
    out = pl.pallas_call(
        _final_body,
        out_shape=jax.ShapeDtypeStruct((N, D_IN), jnp.float32),
    )(acc2, ht2, dis_flat, W2, b2)
    return out


# speedup vs baseline: 1.4538x; 1.0708x over previous
import functools

import jax
import jax.numpy as jnp
from jax import lax
from jax.experimental import pallas as pl
from jax.experimental.pallas import tpu as pltpu
from jax.experimental.pallas import tpu_sc as plsc

N = 10000
E = 320000
D_IN = 128
D_HID = 16
NC = 2
NS = 16
CHUNK = 128
GCH = E // CHUNK
PCH = GCH // NC
WCH = PCH // NS
PREM = PCH - NS * WCH
DCH = GCH // NS
DREM = GCH - NS * DCH
NP = 10240
RPW = NP // NS
NBUF, LAG = 10, 5

_mesh = plsc.VectorSubcoreMesh(
    core_axis_name="c", subcore_axis_name="s", num_cores=NC, num_subcores=NS)

_SC_PARAMS = pltpu.CompilerParams(
    use_tc_tiling_on_sc=False, needs_layout_passes=False)


def _rsqrt16(d):
    i = plsc.bitcast(d, jnp.int32)
    y = plsc.bitcast(jnp.int32(0x5F3759DF) - (i >> 1), jnp.float32)
    for _ in range(3):
        y = y * (1.5 - 0.5 * d * y * y)
    return y


def _fill_rows(ref, rows):
    val16 = jnp.zeros((16,), jnp.float32)

    def body(i, _):
        ref[i, :] = val16
        return 0
    lax.fori_loop(0, rows, body, 0)


def _edge_pass(ht_sh, src_v, dst_v, rows_v, acc_sh, gsems, ssems, extra):
    gd = [None] * WCH
    sd = [None] * WCH
    for j in range(WCH):
        b = j % NBUF
        if j >= NBUF:
            sd[j - NBUF].wait()
        gd[j] = pltpu.async_copy(ht_sh.at[src_v.at[j]], rows_v.at[b],
                                 gsems[b])
        if j >= LAG:
            k = j - LAG
            gd[k].wait()
            sd[k] = pltpu.async_copy(rows_v.at[k % NBUF],
                                     acc_sh.at[dst_v.at[k]],
                                     ssems[k % NBUF], add=True)
    for k in range(WCH - LAG, WCH):
        gd[k].wait()
        sd[k] = pltpu.async_copy(rows_v.at[k % NBUF],
                                 acc_sh.at[dst_v.at[k]],
                                 ssems[k % NBUF], add=True)
    for k in range(WCH - NBUF, WCH):
        sd[k].wait()

    @pl.when(extra)
    def _():
        pltpu.sync_copy(ht_sh.at[src_v.at[WCH]], rows_v.at[0])
        pltpu.sync_copy(rows_v.at[0], acc_sh.at[dst_v.at[WCH]], add=True)


def _load_worker_chunks(src_hbm, dst_hbm, src_v, dst_v, cid, sid):
    base = cid * PCH + sid * WCH
    xrow = cid * PCH + NS * WCH + lax.rem(sid, PREM)
    pltpu.sync_copy(src_hbm.at[pl.ds(base, WCH)], src_v.at[pl.ds(0, WCH)])
    pltpu.sync_copy(dst_hbm.at[pl.ds(base, WCH)], dst_v.at[pl.ds(0, WCH)])
    pltpu.sync_copy(src_hbm.at[pl.ds(xrow, 1)], src_v.at[pl.ds(WCH, 1)])
    pltpu.sync_copy(dst_hbm.at[pl.ds(xrow, 1)], dst_v.at[pl.ds(WCH, 1)])


@functools.partial(
    pl.kernel,
    out_type=(jax.ShapeDtypeStruct((NC, NP, D_HID), jnp.float32),
              jax.ShapeDtypeStruct((NC * NP, D_HID), jnp.float32),
              jax.ShapeDtypeStruct((NC * NP // 16, 16), jnp.float32)),
    mesh=_mesh,
    scratch_types=[
        pltpu.VMEM((DCH + 1, CHUNK), jnp.int32),
        pltpu.VMEM((WCH + 1, CHUNK), jnp.int32),
        pltpu.VMEM((WCH + 1, CHUNK), jnp.int32),
        pltpu.VMEM((RPW, D_HID), jnp.float32),
        pltpu.VMEM((RPW,), jnp.float32),
        pltpu.VMEM((RPW // 16, 16), jnp.float32),
        pltpu.VMEM((RPW,), jnp.float32),
        pltpu.VMEM((CHUNK,), jnp.float32),
        pltpu.VMEM((CHUNK, D_HID), jnp.float32),
        pltpu.VMEM((NBUF, CHUNK, D_HID), jnp.float32),
        pltpu.VMEM_SHARED((NP,), jnp.float32),
        pltpu.VMEM_SHARED((NP, D_HID), jnp.float32),
        pltpu.VMEM_SHARED((NP, D_HID), jnp.float32),
        [pltpu.SemaphoreType.DMA] * 8,
        [pltpu.SemaphoreType.DMA] * NBUF,
        [pltpu.SemaphoreType.DMA] * NBUF,
        pltpu.SemaphoreType.DMA,
        pltpu.SemaphoreType.DMA,
    ],
    compiler_params=_SC_PARAMS,
)
def _sc1_kernel(h1_hbm, src_hbm, dst_hbm, acc_out, ht_out, dis_out,
                ddst_v, src_v, dst_v, lrows_v, ldeg_v, ldis_v, zrow_v,
                ones_v, ztile_v, rows_v, deg_sh, acc_sh, ht_sh, dsems,
                gsems, ssems, hsem, hsem2):
    cid = lax.axis_index("c")
    sid = lax.axis_index("s")
    nbase = sid * RPW

    hdesc = pltpu.async_copy(h1_hbm.at[pl.ds(nbase, RPW)], lrows_v, hsem)
    pltpu.sync_copy(dst_hbm.at[pl.ds(sid * DCH, DCH)],
                    ddst_v.at[pl.ds(0, DCH)])
    pltpu.sync_copy(dst_hbm.at[pl.ds(NS * DCH + lax.rem(sid, DREM), 1)],
                    ddst_v.at[pl.ds(DCH, 1)])
    _load_worker_chunks(src_hbm, dst_hbm, src_v, dst_v, cid, sid)

    one16 = jnp.ones((16,), jnp.float32)
    for i in range(CHUNK // 16):
        ones_v[pl.ds(i * 16, 16)] = one16
    for g in range(RPW // 16):
        zrow_v[pl.ds(g * 16, 16)] = jnp.zeros((16,), jnp.float32)
    _fill_rows(ztile_v, CHUNK)
    pltpu.sync_copy(zrow_v, deg_sh.at[pl.ds(nbase, RPW)])
    for t in range(RPW // CHUNK):
        pltpu.sync_copy(ztile_v, acc_sh.at[pl.ds(nbase + t * CHUNK, CHUNK)])
    plsc.subcore_barrier()

    dd = [None] * DCH
    for j in range(DCH):
        if j >= 8:
            dd[j - 8].wait()
        dd[j] = pltpu.async_copy(ones_v, deg_sh.at[ddst_v.at[j]],
                                 dsems[j % 8], add=True)
    for j in range(DCH - 8, DCH):
        dd[j].wait()

    @pl.when(sid < DREM)
    def _():
        pltpu.sync_copy(ones_v, deg_sh.at[ddst_v.at[DCH]], add=True)
    plsc.subcore_barrier()

    pltpu.sync_copy(deg_sh.at[pl.ds(nbase, RPW)], ldeg_v)
    hdesc.wait()
    for g in range(RPW // 16):
        ldis_v[g, :] = _rsqrt16(ldeg_v[pl.ds(g * 16, 16)] + 1.0)

    def scale_body(g, _):
        dvec = ldis_v[g, :]
        for k in range(16):
            r = g * 16 + k
            lrows_v[r, :] = lrows_v[r, :] * dvec[k]
        return 0
    lax.fori_loop(0, RPW // 16, scale_body, 0)
    pltpu.sync_copy(lrows_v, ht_sh.at[pl.ds(nbase, RPW)])
    hd2 = pltpu.async_copy(lrows_v, ht_out.at[pl.ds(cid * NP + nbase, RPW)],
                           hsem2)
    pltpu.sync_copy(
        ldis_v, dis_out.at[pl.ds(cid * (NP // 16) + sid * (RPW // 16),
                                 RPW // 16)])
    plsc.subcore_barrier()

    _edge_pass(ht_sh, src_v, dst_v, rows_v, acc_sh, gsems, ssems,
               sid < PREM)
    hd2.wait()
    plsc.subcore_barrier()
    pltpu.sync_copy(acc_sh.at[pl.ds(nbase, RPW)],
                    acc_out.at[cid, pl.ds(nbase, RPW)])


@functools.partial(
    pl.kernel,
    out_type=jax.ShapeDtypeStruct((NC, NP, D_HID), jnp.float32),
    mesh=_mesh,
    scratch_types=[
        pltpu.VMEM((WCH + 1, CHUNK), jnp.int32),
        pltpu.VMEM((WCH + 1, CHUNK), jnp.int32),
        pltpu.VMEM((RPW, D_HID), jnp.float32),
        pltpu.VMEM((RPW, D_HID), jnp.float32),
        pltpu.VMEM((RPW, D_HID), jnp.float32),
        pltpu.VMEM((RPW // 16, 16), jnp.float32),
        pltpu.VMEM((16,), jnp.float32),
        pltpu.VMEM((CHUNK, D_HID), jnp.float32),
        pltpu.VMEM((NBUF, CHUNK, D_HID), jnp.float32),
        pltpu.VMEM_SHARED((NP, D_HID), jnp.float32),
        pltpu.VMEM_SHARED((NP, D_HID), jnp.float32),
        [pltpu.SemaphoreType.DMA] * NBUF,
        [pltpu.SemaphoreType.DMA] * NBUF,
    ],
    compiler_params=_SC_PARAMS,
)
def _sc2_kernel(acc1_hbm, ht1_hbm, dis_hbm, b1_hbm, src_hbm, dst_hbm,
                z_out,
                src_v, dst_v, lrows_v, lacc_v, lacc2_v, ldis_v, b1_v,
                ztile_v, rows_v, acc_sh, ht_sh, gsems, ssems):
    cid = lax.axis_index("c")
    sid = lax.axis_index("s")
    nbase = sid * RPW

    _load_worker_chunks(src_hbm, dst_hbm, src_v, dst_v, cid, sid)
    pltpu.sync_copy(acc1_hbm.at[0, pl.ds(nbase, RPW)], lacc_v)
    pltpu.sync_copy(acc1_hbm.at[1, pl.ds(nbase, RPW)], lacc2_v)
    pltpu.sync_copy(ht1_hbm.at[pl.ds(cid * NP + nbase, RPW)], lrows_v)
    pltpu.sync_copy(
        dis_hbm.at[pl.ds(cid * (NP // 16) + sid * (RPW // 16), RPW // 16)],
        ldis_v)
    pltpu.sync_copy(b1_hbm, b1_v)
    _fill_rows(ztile_v, CHUNK)
    for t in range(RPW // CHUNK):
        pltpu.sync_copy(ztile_v, acc_sh.at[pl.ds(nbase + t * CHUNK, CHUNK)])

    b1vec = b1_v[...]

    def relu_body(g, _):
        dvec = ldis_v[g, :]
        for k in range(16):
            r = g * 16 + k
            s = dvec[k]
            hr = jnp.maximum(
                s * (lacc_v[r, :] + lacc2_v[r, :] + lrows_v[r, :]) + b1vec,
                0.0)
            lrows_v[r, :] = s * hr
        return 0
    lax.fori_loop(0, RPW // 16, relu_body, 0)
    pltpu.sync_copy(lrows_v, ht_sh.at[pl.ds(nbase, RPW)])
    plsc.subcore_barrier()

    _edge_pass(ht_sh, src_v, dst_v, rows_v, acc_sh, gsems, ssems,
               sid < PREM)
    plsc.subcore_barrier()

    pltpu.sync_copy(acc_sh.at[pl.ds(nbase, RPW)], lacc_v)

    def z_body(g, _):
        dvec = ldis_v[g, :]
        for k in range(16):
            r = g * 16 + k
            lacc_v[r, :] = dvec[k] * (lacc_v[r, :] + 0.5 * lrows_v[r, :])
        return 0
    lax.fori_loop(0, RPW // 16, z_body, 0)
    pltpu.sync_copy(lacc_v, z_out.at[cid, pl.ds(nbase, RPW)])


def _mm1_body(x_ref, w1_ref, h_ref):
    h_ref[:N, :] = jnp.dot(x_ref[...], w1_ref[...],
                           preferred_element_type=jnp.float32)
    h_ref[N:, :] = jnp.zeros((NP - N, D_HID), jnp.float32)


def _final_body(z_ref, w2_ref, b2_ref, out_ref):
    agg = z_ref[0] + z_ref[1]
    out_ref[...] = (
        jnp.dot(agg[:N], w2_ref[...], preferred_element_type=jnp.float32)
        + b2_ref[...][None, :])


def kernel(x, edge_index, W1, b1, W2, b2):
    src2 = edge_index[0].reshape(GCH, CHUNK)
    dst2 = edge_index[1].reshape(GCH, CHUNK)

    h1 = pl.pallas_call(
        _mm1_body,
        out_shape=jax.ShapeDtypeStruct((NP, D_HID), jnp.float32),
    )(x, W1)

    acc1, ht1, dis = _sc1_kernel(h1, src2, dst2)
    z = _sc2_kernel(acc1, ht1, dis, b1, src2, dst2)

    out = pl.pallas_call(
        _final_body,
        out_shape=jax.ShapeDtypeStruct((N, D_IN), jnp.float32),
    )(z, W2, b2)
    return out
